# Initial kernel scaffold; baseline (speedup 1.0000x reference)
import jax
import jax.numpy as jnp
from jax.experimental import pallas as pl


def kernel(hidden_states, Wq, bq, Wk, bk, Wv, bv, Wo, bo, memory_keys, memory_values, memory_age):
    raise NotImplementedError("write your pallas kernel here")


# speedup vs baseline: 1.4820x; 1.4820x over previous
import math

import jax
import jax.numpy as jnp
from jax.experimental import pallas as pl
from jax.experimental.pallas import tpu as pltpu

H = 512
M = 65536
B = 8
BLK = 2048
NBLK = M // BLK
SCALE = 1.0 / math.sqrt(float(H))


def _qkv_body(hs_ref, wq_ref, bq_ref, wk_ref, bk_ref, wv_ref, bv_ref,
              q_ref, k_ref, v_ref):
    hs = hs_ref[...]

    def proj(w_ref, b_ref):
        return jax.lax.dot_general(
            hs, w_ref[...], (((1,), (1,)), ((), ())),
            preferred_element_type=jnp.float32) + b_ref[...]

    q_ref[...] = proj(wq_ref, bq_ref) * SCALE
    k_ref[...] = proj(wk_ref, bk_ref)
    v_ref[...] = proj(wv_ref, bv_ref)


def _attn_body(q_ref, k_ref, v_ref,
               newk_ref, newv_ref, scores_ref, acc_out_ref,
               m_s, l_s, acc_s):
    i = pl.program_id(0)

    @pl.when(i == 0)
    def _init():
        m_s[...] = jnp.full_like(m_s, -1e30)
        l_s[...] = jnp.zeros_like(l_s)
        acc_s[...] = jnp.zeros_like(acc_s)

    k = k_ref[...]
    v = v_ref[...]
    newk_ref[...] = k
    newv_ref[...] = v
    s = jax.lax.dot_general(q_ref[...], k, (((1,), (1,)), ((), ())),
                            preferred_element_type=jnp.float32)
    scores_ref[...] = s

    m_old = m_s[:, :1]
    l_old = l_s[:, :1]
    m_new = jnp.maximum(m_old, jnp.max(s, axis=1, keepdims=True))
    p = jnp.exp(s - m_new)
    alpha = jnp.exp(m_old - m_new)
    l_new = l_old * alpha + jnp.sum(p, axis=1, keepdims=True)
    acc_s[...] = acc_s[...] * alpha + jax.lax.dot_general(
        p, v, (((1,), (0,)), ((), ())), preferred_element_type=jnp.float32)
    m_s[...] = jnp.broadcast_to(m_new, m_s.shape)
    l_s[...] = jnp.broadcast_to(l_new, l_s.shape)

    @pl.when(i == NBLK - 1)
    def _fin():
        acc_out_ref[...] = acc_s[...]


def _epi_body(scores_ref, acc_ref, age_ref, wo_ref, bo_ref,
              out_ref, cnt_ref, newage_ref, maxsc_ref, usage_ref, idx_ref):
    s = scores_ref[...]
    m = jnp.max(s, axis=1, keepdims=True)
    e = jnp.exp(s - m)
    l = jnp.sum(e, axis=1, keepdims=True)
    probs = e / l
    imp = jnp.sum(probs, axis=0, keepdims=True)
    cnt_ref[...] = jnp.sum((probs > 0.01).astype(jnp.int32), axis=0,
                           keepdims=True)

    age = age_ref[...]
    t = age + 2.0 - imp
    maxt = jnp.max(t)
    iota = jax.lax.broadcasted_iota(jnp.int32, t.shape, 1)
    idx = jnp.min(jnp.where(t == maxt, iota, M))
    idx_ref[...] = jnp.full((1, 1), idx, jnp.int32)

    new_age = jnp.where(iota == idx, 0.0, age + 1.0)
    newage_ref[...] = new_age
    maxsc_ref[...] = jnp.mean(jnp.max(s, axis=1)).reshape(1, 1)
    usage_ref[...] = jnp.mean((new_age > 0.0).astype(jnp.float32)).reshape(1, 1)

    o = acc_ref[...] / l
    out_ref[...] = jax.lax.dot_general(
        o, wo_ref[...], (((1,), (1,)), ((), ())),
        preferred_element_type=jnp.float32) + bo_ref[...]


def _scatter_body(idx_ref, krow_ref, vrow_ref, keys_in_ref, vals_in_ref,
                  keys_out_ref, vals_out_ref, sem):
    del keys_in_ref, vals_in_ref
    i = idx_ref[0, 0]
    ck = pltpu.make_async_copy(krow_ref, keys_out_ref.at[pl.ds(i, 1), :], sem)
    ck.start()
    ck.wait()
    cv = pltpu.make_async_copy(vrow_ref, vals_out_ref.at[pl.ds(i, 1), :], sem)
    cv.start()
    cv.wait()


def kernel(hidden_states, Wq, bq, Wk, bk, Wv, bv, Wo, bo,
           memory_keys, memory_values, memory_age):
    f32 = jnp.float32
    hs = hidden_states.reshape(B, H)
    mk = memory_keys.reshape(M, H)
    mv = memory_values.reshape(M, H)

    q, k, v = pl.pallas_call(
        _qkv_body,
        out_shape=[jax.ShapeDtypeStruct((B, H), f32)] * 3,
    )(hs, Wq, bq.reshape(1, H), Wk, bk.reshape(1, H), Wv, bv.reshape(1, H))

    new_k, new_v, scores, acc = pl.pallas_call(
        _attn_body,
        grid=(NBLK,),
        in_specs=[
            pl.BlockSpec((B, H), lambda i: (0, 0)),
            pl.BlockSpec((BLK, H), lambda i: (i, 0)),
            pl.BlockSpec((BLK, H), lambda i: (i, 0)),
        ],
        out_specs=[
            pl.BlockSpec((BLK, H), lambda i: (i, 0)),
            pl.BlockSpec((BLK, H), lambda i: (i, 0)),
            pl.BlockSpec((B, BLK), lambda i: (0, i)),
            pl.BlockSpec((B, H), lambda i: (0, 0)),
        ],
        out_shape=[
            jax.ShapeDtypeStruct((M, H), f32),
            jax.ShapeDtypeStruct((M, H), f32),
            jax.ShapeDtypeStruct((B, M), f32),
            jax.ShapeDtypeStruct((B, H), f32),
        ],
        scratch_shapes=[
            pltpu.VMEM((B, 128), f32),
            pltpu.VMEM((B, 128), f32),
            pltpu.VMEM((B, H), f32),
        ],
    )(q, mk, mv)

    out_p, cnt, new_age, maxsc, usage, idx = pl.pallas_call(
        _epi_body,
        out_shape=[
            jax.ShapeDtypeStruct((B, H), f32),
            jax.ShapeDtypeStruct((1, M), jnp.int32),
            jax.ShapeDtypeStruct((1, M), f32),
            jax.ShapeDtypeStruct((1, 1), f32),
            jax.ShapeDtypeStruct((1, 1), f32),
            jax.ShapeDtypeStruct((1, 1), jnp.int32),
        ],
    )(scores, acc, memory_age, Wo, bo.reshape(1, H))

    keys_f, vals_f = pl.pallas_call(
        _scatter_body,
        in_specs=[
            pl.BlockSpec(memory_space=pltpu.SMEM),
            pl.BlockSpec(memory_space=pltpu.VMEM),
            pl.BlockSpec(memory_space=pltpu.VMEM),
            pl.BlockSpec(memory_space=pl.ANY),
            pl.BlockSpec(memory_space=pl.ANY),
        ],
        out_specs=[
            pl.BlockSpec(memory_space=pl.ANY),
            pl.BlockSpec(memory_space=pl.ANY),
        ],
        out_shape=[
            jax.ShapeDtypeStruct((M, H), f32),
            jax.ShapeDtypeStruct((M, H), f32),
        ],
        input_output_aliases={3: 0, 4: 1},
        scratch_shapes=[pltpu.SemaphoreType.DMA],
    )(idx, k[0:1], v[0:1], new_k, new_v)

    return (out_p.reshape(B, 1, H),
            cnt,
            maxsc.reshape(()),
            usage.reshape(()),
            keys_f.reshape(1, M, H),
            vals_f.reshape(1, M, H),
            new_age)
